# Initial kernel scaffold; baseline (speedup 1.0000x reference)
import jax
import jax.numpy as jnp
from jax.experimental import pallas as pl


def kernel(x, W, b):
    raise NotImplementedError("write your pallas kernel here")


# speedup vs baseline: 5.0328x; 5.0328x over previous
import jax
import jax.numpy as jnp
from jax.experimental import pallas as pl
from jax.experimental.pallas import tpu as pltpu

_TOKENS = 32768
_D = 4096
_E = 64
_K = 8
_BT = 512


def _router_block(x_ref, wt_ref, b_ref, idx_ref, probs_ref):
    xb = x_ref[...]
    logits = (
        jnp.dot(xb, wt_ref[...], preferred_element_type=jnp.float32)
        + b_ref[...]
    )
    iota = jax.lax.broadcasted_iota(jnp.int32, logits.shape, 1)
    neg = jnp.float32(-jnp.inf)
    work = logits
    sel = jnp.zeros(logits.shape, jnp.bool_)
    idx_cols = []
    m0 = None
    for k in range(_K):
        m = jnp.max(work, axis=1, keepdims=True)
        if k == 0:
            m0 = m
        idxk = jnp.min(jnp.where(work == m, iota, _E), axis=1, keepdims=True)
        chosen = iota == idxk
        sel = jnp.logical_or(sel, chosen)
        work = jnp.where(chosen, neg, work)
        idx_cols.append(idxk)
    idx_ref[...] = jnp.concatenate(idx_cols, axis=1)
    e = jnp.where(sel, jnp.exp(logits - m0), jnp.float32(0.0))
    probs_ref[...] = e / jnp.sum(e, axis=1, keepdims=True)


def kernel(x, W, b):
    wt = W.T
    b2 = b.reshape(1, _E)
    grid = (_TOKENS // _BT,)
    idx, probs = pl.pallas_call(
        _router_block,
        grid=grid,
        in_specs=[
            pl.BlockSpec((_BT, _D), lambda i: (i, 0)),
            pl.BlockSpec((_D, _E), lambda i: (0, 0)),
            pl.BlockSpec((1, _E), lambda i: (0, 0)),
        ],
        out_specs=[
            pl.BlockSpec((_BT, _K), lambda i: (i, 0)),
            pl.BlockSpec((_BT, _E), lambda i: (i, 0)),
        ],
        out_shape=[
            jax.ShapeDtypeStruct((_TOKENS, _K), jnp.int32),
            jax.ShapeDtypeStruct((_TOKENS, _E), jnp.float32),
        ],
        compiler_params=pltpu.CompilerParams(
            dimension_semantics=("arbitrary",),
        ),
    )(x, wt, b2)
    return idx, probs


# speedup vs baseline: 5.8209x; 1.1566x over previous
import jax
import jax.numpy as jnp
from jax.experimental import pallas as pl
from jax.experimental.pallas import tpu as pltpu

_TOKENS = 32768
_D = 4096
_E = 64
_K = 8
_BT = 1024


def _router_block(x_ref, wt_ref, b_ref, idx_ref, probs_ref):
    xb = x_ref[...]
    logits = (
        jnp.dot(xb, wt_ref[...], preferred_element_type=jnp.float32)
        + b_ref[...]
    )
    iota = jax.lax.broadcasted_iota(jnp.int32, logits.shape, 1)
    neg = jnp.float32(-jnp.inf)
    work = logits
    sel = jnp.zeros(logits.shape, jnp.bool_)
    idx_cols = []
    m0 = None
    for k in range(_K):
        m = jnp.max(work, axis=1, keepdims=True)
        if k == 0:
            m0 = m
        idxk = jnp.min(jnp.where(work == m, iota, _E), axis=1, keepdims=True)
        chosen = iota == idxk
        sel = jnp.logical_or(sel, chosen)
        work = jnp.where(chosen, neg, work)
        idx_cols.append(idxk)
    idx_ref[...] = jnp.concatenate(idx_cols, axis=1)
    e = jnp.where(sel, jnp.exp(logits - m0), jnp.float32(0.0))
    probs_ref[...] = e / jnp.sum(e, axis=1, keepdims=True)


def kernel(x, W, b):
    wt = W.T
    b2 = b.reshape(1, _E)
    grid = (_TOKENS // _BT,)
    idx, probs = pl.pallas_call(
        _router_block,
        grid=grid,
        in_specs=[
            pl.BlockSpec((_BT, _D), lambda i: (i, 0)),
            pl.BlockSpec((_D, _E), lambda i: (0, 0)),
            pl.BlockSpec((1, _E), lambda i: (0, 0)),
        ],
        out_specs=[
            pl.BlockSpec((_BT, _K), lambda i: (i, 0)),
            pl.BlockSpec((_BT, _E), lambda i: (i, 0)),
        ],
        out_shape=[
            jax.ShapeDtypeStruct((_TOKENS, _K), jnp.int32),
            jax.ShapeDtypeStruct((_TOKENS, _E), jnp.float32),
        ],
        compiler_params=pltpu.CompilerParams(
            dimension_semantics=("arbitrary",),
        ),
    )(x, wt, b2)
    return idx, probs


# speedup vs baseline: 7.9691x; 1.3691x over previous
import jax
import jax.numpy as jnp
from jax.experimental import pallas as pl
from jax.experimental.pallas import tpu as pltpu

_TOKENS = 32768
_D = 4096
_E = 64
_K = 8
_BT = 1024
_NEG = float("-inf")


def _router_block(x_ref, w_ref, b_ref, idx_ref, probs_ref):
    xb = x_ref[...]
    logits = jax.lax.dot_general(
        w_ref[...], xb, (((1,), (1,)), ((), ())),
        preferred_element_type=jnp.float32,
    ) + b_ref[...]
    iota = jax.lax.broadcasted_iota(jnp.int32, logits.shape, 0)
    work = logits
    idx_rows = []
    m0 = None
    for k in range(_K):
        m = jnp.max(work, axis=0, keepdims=True)
        if k == 0:
            m0 = m
        idxk = jnp.min(jnp.where(work == m, iota, _E), axis=0, keepdims=True)
        chosen = iota == idxk
        work = jnp.where(chosen, _NEG, work)
        idx_rows.append(idxk)
    idx_ref[...] = jnp.concatenate(idx_rows, axis=0)
    sel = work == _NEG
    e = jnp.where(sel, jnp.exp(logits - m0), jnp.float32(0.0))
    probs_ref[...] = e / jnp.sum(e, axis=0, keepdims=True)


def kernel(x, W, b):
    b2 = b.reshape(_E, 1)
    grid = (_TOKENS // _BT,)
    idx_t, probs_t = pl.pallas_call(
        _router_block,
        grid=grid,
        in_specs=[
            pl.BlockSpec((_BT, _D), lambda i: (i, 0)),
            pl.BlockSpec((_E, _D), lambda i: (0, 0)),
            pl.BlockSpec((_E, 1), lambda i: (0, 0)),
        ],
        out_specs=[
            pl.BlockSpec((_K, _BT), lambda i: (0, i)),
            pl.BlockSpec((_E, _BT), lambda i: (0, i)),
        ],
        out_shape=[
            jax.ShapeDtypeStruct((_K, _TOKENS), jnp.int32),
            jax.ShapeDtypeStruct((_E, _TOKENS), jnp.float32),
        ],
        compiler_params=pltpu.CompilerParams(
            dimension_semantics=("arbitrary",),
        ),
    )(x, W, b2)
    return idx_t.T, probs_t.T
